# baseline (speedup 1.0000x reference)
import functools

import jax
import jax.numpy as jnp
from jax import lax
from jax.experimental import layout as jlayout
from jax.experimental import pallas as pl
from jax.experimental.pallas import tpu as pltpu
from jax.experimental.pallas import tpu_sc as plsc

NC = 2
NS = 16
L = 16
NW = NC * NS

SUB = 100


def _sinusoidal_pe(seq_len, d_model):
    pos = jnp.arange(seq_len, dtype=jnp.float32)[:, None]
    div = jnp.exp(
        jnp.arange(0, d_model, 2, dtype=jnp.float32)
        * (-jnp.log(10000.0) / d_model)
    )
    pe = jnp.zeros((seq_len, d_model), dtype=jnp.float32)
    pe = pe.at[:, 0::2].set(jnp.sin(pos * div))
    pe = pe.at[:, 1::2].set(jnp.cos(pos * div))
    return pe


@functools.partial(jax.jit, static_argnames=("batch", "seq", "dim"))
def _embed(idx2d, table, pe, *, batch, seq, dim):
    n_rows = batch * seq
    rows_per_w = n_rows // NW
    chunks = rows_per_w // seq
    spc = seq // SUB
    idx_rows_per_w = rows_per_w // SUB

    mesh = plsc.VectorSubcoreMesh(
        core_axis_name="c", subcore_axis_name="s",
        num_cores=NC, num_subcores=NS,
    )

    @functools.partial(
        pl.kernel,
        out_type=jax.ShapeDtypeStruct((n_rows, dim), jnp.float32),
        mesh=mesh,
        compiler_params=pltpu.CompilerParams(
            use_tc_tiling_on_sc=False, skip_device_barrier=True,
        ),
        scratch_types=[
            pltpu.VMEM((idx_rows_per_w, SUB), jnp.int32),
            pltpu.VMEM((2, seq, dim), jnp.float32),
            pltpu.VMEM((seq, dim), jnp.float32),
            pltpu.SemaphoreType.DMA,
            pltpu.SemaphoreType.DMA,
            pltpu.SemaphoreType.DMA,
            pltpu.SemaphoreType.DMA,
        ],
    )
    def k(table_hbm, idx_hbm, pe_hbm, out_hbm, idx_v, rows_v, pe_v,
          sem_g0, sem_g1, sem_w0, sem_w1):
        sem_g = (sem_g0, sem_g1)
        sem_w = (sem_w0, sem_w1)
        wid = lax.axis_index("s") * NC + lax.axis_index("c")
        row_base = wid * rows_per_w
        idx_base = wid * idx_rows_per_w

        pltpu.sync_copy(idx_hbm.at[pl.ds(idx_base, idx_rows_per_w)], idx_v)
        pltpu.sync_copy(pe_hbm, pe_v)

        def fire_gather(c, p):
            for j in range(spc):
                pltpu.async_copy(
                    table_hbm.at[idx_v.at[c * spc + j]],
                    rows_v.at[p, pl.ds(j * SUB, SUB)],
                    sem_g[p],
                )

        def drain_gather(c, p):
            for j in range(spc):
                pltpu.make_async_copy(
                    table_hbm.at[idx_v.at[c * spc + j]],
                    rows_v.at[p, pl.ds(j * SUB, SUB)],
                    sem_g[p],
                ).wait()

        def pe_add(p):
            def pe_row(r, _):
                for g in range(dim // L):
                    plsc.addupdate(
                        rows_v.at[p, r, pl.ds(g * L, L)],
                        pe_v[r, pl.ds(g * L, L)],
                    )
                return ()

            lax.fori_loop(0, seq, pe_row, (), unroll=4)

        def fire_write(c, p):
            pltpu.async_copy(
                rows_v.at[p],
                out_hbm.at[pl.ds(row_base + c * seq, seq)],
                sem_w[p],
            )

        def wait_write(c, p):
            pltpu.make_async_copy(
                rows_v.at[p],
                out_hbm.at[pl.ds(row_base + c * seq, seq)],
                sem_w[p],
            ).wait()

        fire_gather(0, 0)
        fire_gather(1, 1)
        drain_gather(0, 0)
        pe_add(0)
        fire_write(0, 0)

        def pair_body(c2, _):
            for parity in (1, 0):
                c = 2 * c2 + (1 if parity == 1 else 2)
                drain_gather(c, parity)
                pe_add(parity)
                fire_write(c, parity)
                wait_write(c - 1, 1 - parity)
                fire_gather(c + 1, 1 - parity)
            return ()

        lax.fori_loop(0, (chunks - 2) // 2, pair_body, ())

        c_last = chunks - 1
        drain_gather(c_last, 1)
        pe_add(1)
        fire_write(c_last, 1)
        wait_write(c_last - 1, 0)
        wait_write(c_last, 1)

    return k(table, idx2d, pe)


def kernel(input, table):
    batch, seq = input.shape
    dim = table.shape[1]
    idx2d = input.reshape(-1, SUB).astype(jnp.int32)
    pe = _sinusoidal_pe(seq, dim)
    out = _embed(idx2d, table, pe, batch=batch, seq=seq, dim=dim)
    return out.reshape(batch, seq, dim)


# speedup vs baseline: 1.2188x; 1.2188x over previous
import functools

import jax
import jax.numpy as jnp
from jax import lax
from jax.experimental import layout as jlayout
from jax.experimental import pallas as pl
from jax.experimental.pallas import tpu as pltpu
from jax.experimental.pallas import tpu_sc as plsc

NC = 2
NS = 16
L = 16
NW = NC * NS

SUB = 100


def _sinusoidal_pe(seq_len, d_model):
    pos = jnp.arange(seq_len, dtype=jnp.float32)[:, None]
    div = jnp.exp(
        jnp.arange(0, d_model, 2, dtype=jnp.float32)
        * (-jnp.log(10000.0) / d_model)
    )
    pe = jnp.zeros((seq_len, d_model), dtype=jnp.float32)
    pe = pe.at[:, 0::2].set(jnp.sin(pos * div))
    pe = pe.at[:, 1::2].set(jnp.cos(pos * div))
    return pe


@functools.partial(jax.jit, static_argnames=("batch", "seq", "dim"))
def _embed(idx2d, table, pe, *, batch, seq, dim):
    n_rows = batch * seq
    rows_per_w = n_rows // NW
    chunks = rows_per_w // seq
    spc = seq // SUB
    idx_rows_per_w = rows_per_w // SUB

    mesh = plsc.VectorSubcoreMesh(
        core_axis_name="c", subcore_axis_name="s",
        num_cores=NC, num_subcores=NS,
    )

    @functools.partial(
        pl.kernel,
        out_type=jax.ShapeDtypeStruct((n_rows, dim), jnp.float32),
        mesh=mesh,
        compiler_params=pltpu.CompilerParams(
            use_tc_tiling_on_sc=False, skip_device_barrier=True,
        ),
        scratch_types=[
            pltpu.VMEM((idx_rows_per_w, SUB), jnp.int32),
            pltpu.VMEM((2, seq, dim), jnp.float32),
            pltpu.VMEM((seq, dim), jnp.float32),
            pltpu.SemaphoreType.DMA,
            pltpu.SemaphoreType.DMA,
            pltpu.SemaphoreType.DMA,
            pltpu.SemaphoreType.DMA,
        ],
    )
    def k(table_hbm, idx_hbm, pe_hbm, out_hbm, idx_v, rows_v, pe_v,
          sem_g0, sem_g1, sem_w0, sem_w1):
        sem_g = (sem_g0, sem_g1)
        sem_w = (sem_w0, sem_w1)
        wid = lax.axis_index("s") * NC + lax.axis_index("c")
        row_base = wid * rows_per_w
        idx_base = wid * idx_rows_per_w

        pltpu.sync_copy(idx_hbm.at[pl.ds(idx_base, idx_rows_per_w)], idx_v)
        pltpu.sync_copy(pe_hbm, pe_v)

        def fire_gather(c, p):
            for j in range(spc):
                pltpu.async_copy(
                    table_hbm.at[idx_v.at[c * spc + j]],
                    rows_v.at[p, pl.ds(j * SUB, SUB)],
                    sem_g[p],
                )

        def drain_gather(c, p):
            for j in range(spc):
                pltpu.make_async_copy(
                    table_hbm.at[idx_v.at[c * spc + j]],
                    rows_v.at[p, pl.ds(j * SUB, SUB)],
                    sem_g[p],
                ).wait()

        def pe_add(p):
            def pe_row(r, _):
                for g in range(dim // L):
                    plsc.addupdate(
                        rows_v.at[p, r, pl.ds(g * L, L)],
                        pe_v[r, pl.ds(g * L, L)],
                    )
                return ()

            lax.fori_loop(0, seq, pe_row, (), unroll=4)

        def fire_write(c, p):
            pltpu.async_copy(
                rows_v.at[p],
                out_hbm.at[pl.ds(row_base + c * seq, seq)],
                sem_w[p],
            )

        def wait_write(c, p):
            pltpu.make_async_copy(
                rows_v.at[p],
                out_hbm.at[pl.ds(row_base + c * seq, seq)],
                sem_w[p],
            ).wait()

        fire_gather(0, 0)
        fire_gather(1, 1)
        drain_gather(0, 0)
        pe_add(0)
        fire_write(0, 0)

        def pair_body(c2, _):
            for parity in (1, 0):
                c = 2 * c2 + (1 if parity == 1 else 2)
                drain_gather(c, parity)
                pe_add(parity)
                fire_write(c, parity)
                wait_write(c - 1, 1 - parity)
                fire_gather(c + 1, 1 - parity)
            return ()

        lax.fori_loop(0, (chunks - 2) // 2, pair_body, ())

        c_last = chunks - 1
        drain_gather(c_last, 1)
        pe_add(1)
        fire_write(c_last, 1)
        wait_write(c_last - 1, 0)
        wait_write(c_last, 1)

    return k(table, idx2d, pe)


def kernel(input, table):
    batch, seq = input.shape
    dim = table.shape[1]
    table = jlayout.with_layout_constraint(
        table, jlayout.Layout(major_to_minor=(0, 1))
    )
    idx2d = input.reshape(-1, SUB).astype(jnp.int32)
    pe = _sinusoidal_pe(seq, dim)
    out = _embed(idx2d, table, pe, batch=batch, seq=seq, dim=dim)
    return out.reshape(batch, seq, dim)
